# baseline (speedup 1.0000x reference)
import functools

import jax
import jax.numpy as jnp
from jax import lax
from jax.experimental import pallas as pl
from jax.experimental.pallas import tpu as pltpu
from jax.experimental.pallas import tpu_sc as plsc

_VOCAB = 32002
_HIDDEN = 2048
_BATCH = 4
_SEQ = 4096
_NTOK = _BATCH * _SEQ
_NW = 32
_PER_W = _NTOK // _NW
_CHUNK = 8
_NCHUNK = _PER_W // _CHUNK
_NBUF = 4

_mesh = plsc.VectorSubcoreMesh(core_axis_name="c", subcore_axis_name="s")


@functools.partial(
    pl.kernel,
    out_type=jax.ShapeDtypeStruct((_NTOK, _HIDDEN), jnp.float32),
    mesh=_mesh,
    scratch_types=(
        [pltpu.VMEM((_NCHUNK, _CHUNK), jnp.int32)]
        + [pltpu.VMEM((_CHUNK, _HIDDEN), jnp.float32)] * _NBUF
        + [pltpu.SemaphoreType.DMA] * (2 * _NBUF)
    ),
)
def _embed_lookup(table_hbm, idx_hbm, out_hbm, idx_v, *scratch):
    wid = lax.axis_index("s") * 2 + lax.axis_index("c")
    base = wid * _PER_W
    pltpu.sync_copy(idx_hbm.at[wid], idx_v)

    bufs = scratch[:_NBUF]
    gsems = scratch[_NBUF:2 * _NBUF]
    osems = scratch[2 * _NBUF:]

    def gather_start(j, b):
        pltpu.async_copy(table_hbm.at[idx_v.at[j]], bufs[b], gsems[b])

    def gather_wait(b):
        pltpu.make_async_copy(
            table_hbm.at[idx_v.at[0]], bufs[b], gsems[b]).wait()

    def out_start(j, b):
        pltpu.async_copy(
            bufs[b], out_hbm.at[pl.ds(base + j * _CHUNK, _CHUNK)], osems[b])

    def out_wait(b):
        pltpu.make_async_copy(
            bufs[b], out_hbm.at[pl.ds(base, _CHUNK)], osems[b]).wait()

    for b in range(_NBUF):
        gather_start(b, b)
    for b in range(_NBUF):
        gather_wait(b)
        out_start(b, b)

    def step(k, carry):
        for b in range(_NBUF):
            j = _NBUF * k + b
            out_wait(b)
            gather_start(j, b)
            gather_wait(b)
            out_start(j, b)
        return carry

    lax.fori_loop(1, _NCHUNK // _NBUF, step, 0)
    for b in range(_NBUF):
        out_wait(b)


def kernel(input_ids, embedding_table):
    ids = input_ids.reshape(_NW, _NCHUNK, _CHUNK)
    out = _embed_lookup(embedding_table, ids)
    return out.reshape(_BATCH, _SEQ, _HIDDEN)


# speedup vs baseline: 1.1882x; 1.1882x over previous
import functools

import jax
import jax.numpy as jnp
from jax import lax
from jax.experimental import pallas as pl
from jax.experimental.pallas import tpu as pltpu
from jax.experimental.pallas import tpu_sc as plsc

_VOCAB = 32002
_HIDDEN = 2048
_BATCH = 4
_SEQ = 4096
_NTOK = _BATCH * _SEQ
_NW = 32
_PER_W = _NTOK // _NW
_CHUNK = 16
_NCHUNK = _PER_W // _CHUNK
_NBUF = 2

_mesh = plsc.VectorSubcoreMesh(core_axis_name="c", subcore_axis_name="s")


@functools.partial(
    pl.kernel,
    out_type=jax.ShapeDtypeStruct((_NTOK, _HIDDEN), jnp.float32),
    mesh=_mesh,
    scratch_types=(
        [pltpu.VMEM((_NCHUNK, _CHUNK), jnp.int32)]
        + [pltpu.VMEM((_CHUNK, _HIDDEN), jnp.float32)] * _NBUF
        + [pltpu.SemaphoreType.DMA] * (2 * _NBUF)
    ),
)
def _embed_lookup(table_hbm, idx_hbm, out_hbm, idx_v, *scratch):
    wid = lax.axis_index("s") * 2 + lax.axis_index("c")
    base = wid * _PER_W
    pltpu.sync_copy(idx_hbm.at[wid], idx_v)

    bufs = scratch[:_NBUF]
    gsems = scratch[_NBUF:2 * _NBUF]
    osems = scratch[2 * _NBUF:]

    def gather_start(j, b):
        pltpu.async_copy(table_hbm.at[idx_v.at[j]], bufs[b], gsems[b])

    def gather_wait(b):
        pltpu.make_async_copy(
            table_hbm.at[idx_v.at[0]], bufs[b], gsems[b]).wait()

    def out_start(j, b):
        pltpu.async_copy(
            bufs[b], out_hbm.at[pl.ds(base + j * _CHUNK, _CHUNK)], osems[b])

    def out_wait(b):
        pltpu.make_async_copy(
            bufs[b], out_hbm.at[pl.ds(base, _CHUNK)], osems[b]).wait()

    for b in range(_NBUF):
        gather_start(b, b)
    for b in range(_NBUF):
        gather_wait(b)
        out_start(b, b)

    def step(k, carry):
        for b in range(_NBUF):
            j = _NBUF * k + b
            out_wait(b)
            gather_start(j, b)
            gather_wait(b)
            out_start(j, b)
        return carry

    lax.fori_loop(1, _NCHUNK // _NBUF, step, 0)
    for b in range(_NBUF):
        out_wait(b)


def kernel(input_ids, embedding_table):
    ids = input_ids.reshape(_NW, _NCHUNK, _CHUNK)
    out = _embed_lookup(embedding_table, ids)
    return out.reshape(_BATCH, _SEQ, _HIDDEN)


# speedup vs baseline: 1.2034x; 1.0128x over previous
import functools

import jax
import jax.numpy as jnp
from jax import lax
from jax.experimental import pallas as pl
from jax.experimental.pallas import tpu as pltpu
from jax.experimental.pallas import tpu_sc as plsc

_VOCAB = 32002
_HIDDEN = 2048
_BATCH = 4
_SEQ = 4096
_NTOK = _BATCH * _SEQ
_NW = 32
_PER_W = _NTOK // _NW
_CHUNK = 16
_NCHUNK = _PER_W // _CHUNK
_NSLOT = _NCHUNK + 1
_NBUF = 3

_mesh = plsc.VectorSubcoreMesh(core_axis_name="c", subcore_axis_name="s")


@functools.partial(
    pl.kernel,
    out_type=jax.ShapeDtypeStruct((_NTOK, _HIDDEN), jnp.float32),
    mesh=_mesh,
    scratch_types=(
        [pltpu.VMEM((_NSLOT, _CHUNK), jnp.int32)]
        + [pltpu.VMEM((_CHUNK, _HIDDEN), jnp.float32)] * _NBUF
        + [pltpu.SemaphoreType.DMA] * (2 * _NBUF)
    ),
)
def _embed_lookup(table_hbm, idx_hbm, out_hbm, idx_v, *scratch):
    wid = lax.axis_index("s") * 2 + lax.axis_index("c")
    base = wid * _PER_W
    pltpu.sync_copy(idx_hbm.at[wid], idx_v)

    bufs = scratch[:_NBUF]
    gsems = scratch[_NBUF:2 * _NBUF]
    osems = scratch[2 * _NBUF:]

    def gather_start(j, b):
        pltpu.async_copy(table_hbm.at[idx_v.at[j]], bufs[b], gsems[b])

    def gather_wait(b):
        pltpu.make_async_copy(
            table_hbm.at[idx_v.at[0]], bufs[b], gsems[b]).wait()

    def out_start(j, b):
        pltpu.async_copy(
            bufs[b], out_hbm.at[pl.ds(base + j * _CHUNK, _CHUNK)], osems[b])

    def out_wait(b):
        pltpu.make_async_copy(
            bufs[b], out_hbm.at[pl.ds(base, _CHUNK)], osems[b]).wait()

    gather_start(0, 0)
    gather_start(1, 1)
    gather_wait(0)
    out_start(0, 0)
    gather_start(2, 2)
    gather_wait(1)
    out_start(1, 1)

    def step(k, carry):
        for p in range(_NBUF):
            j = _NBUF * k + p
            b = p
            out_wait(b)
            gather_start(j, b)
            bp = (p - 1) % _NBUF
            gather_wait(bp)
            out_start(j - 1, bp)
        return carry

    lax.fori_loop(1, _NSLOT // _NBUF, step, 0)

    gather_wait(_NCHUNK % _NBUF)
    out_wait((_NCHUNK - 2) % _NBUF)
    out_wait((_NCHUNK - 1) % _NBUF)


def kernel(input_ids, embedding_table):
    ids = input_ids.reshape(_NW, _PER_W)
    ids = jnp.concatenate([ids, ids[:, -_CHUNK:]], axis=1)
    ids = ids.reshape(_NW, _NSLOT, _CHUNK)
    out = _embed_lookup(embedding_table, ids)
    return out.reshape(_BATCH, _SEQ, _HIDDEN)
